# baseline (speedup 1.0000x reference)
import functools

import jax
import jax.numpy as jnp
from jax import lax
from jax.experimental import pallas as pl
from jax.experimental.pallas import tpu as pltpu
from jax.experimental.pallas import tpu_sc as plsc

T = 256
D_IN = 2048
D_OUT = 2048
E = 8
TM = 32
SORT = 512
TO = 1024
NT = D_OUT // TO


def _moe_body(x_ref, gw_ref, gb_ref, w_ref, b_ref, out_ref, aux_ref,
              xs_scr, os_scr, pc_scr, pr_scr, sm_scr):
    n = pl.program_id(0)
    e = pl.program_id(1)

    @pl.when((n == 0) & (e == 0))
    def _route():
        logits = lax.dot_general(
            x_ref[...], gw_ref[...], (((1,), (1,)), ((), ())),
            preferred_element_type=jnp.float32,
        ) + gb_ref[...]
        m = jnp.max(logits, axis=1, keepdims=True)
        unn = jnp.exp(logits - m)
        p = unn / jnp.sum(unn, axis=1, keepdims=True)
        mean_gate = jnp.mean(p, axis=0, keepdims=True)
        aux_ref[...] = jnp.mean((mean_gate * E) ** 2, axis=1, keepdims=True)

        lane = lax.broadcasted_iota(jnp.int32, (T, E), 1)
        is_max = p == jnp.max(p, axis=1, keepdims=True)
        top1 = jnp.min(jnp.where(is_max, lane, E), axis=1, keepdims=True)
        oh = (lane == top1).astype(jnp.float32)

        counts = jnp.sum(oh, axis=0, keepdims=True)
        pc = jnp.ceil(counts / TM) * TM
        re8 = lax.broadcasted_iota(jnp.int32, (E, E), 0)
        ce8 = lax.broadcasted_iota(jnp.int32, (E, E), 1)
        pc_col = jnp.sum(jnp.where(re8 == ce8, pc, 0.0), axis=1,
                         keepdims=True)
        offs = jnp.sum(jnp.where(re8 < ce8, pc_col, 0.0), axis=0,
                       keepdims=True)
        lane1 = lax.broadcasted_iota(jnp.int32, (1, E), 1)
        for ei in range(E):
            ksc = jnp.sum(jnp.where(lane1 == ei, pc, 0.0)) / TM
            osc = jnp.sum(jnp.where(lane1 == ei, offs, 0.0))
            sm_scr[ei] = ksc.astype(jnp.int32)
            sm_scr[E + ei] = osc.astype(jnp.int32)

        rt = lax.broadcasted_iota(jnp.int32, (T, T), 0)
        ct = lax.broadcasted_iota(jnp.int32, (T, T), 1)
        tril = (rt >= ct).astype(jnp.float32)
        pref = lax.dot_general(tril, oh, (((1,), (0,)), ((), ())),
                               preferred_element_type=jnp.float32)
        rank = jnp.sum(pref * oh, axis=1, keepdims=True) - 1.0
        pos = jnp.sum(oh * offs, axis=1, keepdims=True) + rank
        pc_scr[...] = pos.astype(jnp.int32)
        pos_row = jnp.sum(jnp.where(rt == ct, pos, 0.0), axis=0,
                          keepdims=True).astype(jnp.int32)
        pr_scr[...] = pos_row

        r_iota = lax.broadcasted_iota(jnp.int32, (SORT, T), 0)
        p1 = (r_iota == pos_row).astype(jnp.float32)
        xs_scr[...] = lax.dot_general(p1, x_ref[...],
                                      (((1,), (0,)), ((), ())),
                                      preferred_element_type=jnp.float32)
        os_scr[...] = jnp.zeros((SORT, TO), jnp.float32)

    k = sm_scr[e]
    off = sm_scr[E + e]
    r8 = lax.broadcasted_iota(jnp.int32, (E, TO), 0)
    brow = jnp.sum(jnp.where(r8 == e, b_ref[...], 0.0), axis=0, keepdims=True)

    def _subtile(i):
        r0 = pl.multiple_of(off + i * TM, TM)
        xs = xs_scr[pl.ds(r0, TM), :]
        acc = lax.dot_general(xs, w_ref[0], (((1,), (1,)), ((), ())),
                              preferred_element_type=jnp.float32)
        os_scr[pl.ds(r0, TM), :] = acc + brow

    for si in range(3):
        @pl.when(si < k)
        def _static(si=si):
            _subtile(si)

    @pl.when(k > 3)
    def _overflow():
        def mloop(i, _):
            _subtile(i)
            return 0
        lax.fori_loop(3, k, mloop, 0)

    @pl.when(e == E - 1)
    def _combine():
        c_iota = lax.broadcasted_iota(jnp.int32, (T, SORT), 1)
        p2 = (c_iota == pc_scr[...]).astype(jnp.float32)
        out_ref[...] = lax.dot_general(p2, os_scr[...],
                                       (((1,), (0,)), ((), ())),
                                       preferred_element_type=jnp.float32)


def kernel(x, gate_W, gate_b, W, b):
    out, aux = pl.pallas_call(
        _moe_body,
        grid=(NT, E),
        in_specs=[
            pl.BlockSpec((T, D_IN), lambda n, e: (0, 0)),
            pl.BlockSpec((E, D_IN), lambda n, e: (0, 0)),
            pl.BlockSpec((1, E), lambda n, e: (0, 0)),
            pl.BlockSpec((1, TO, D_IN), lambda n, e: (e, n, 0)),
            pl.BlockSpec((E, TO), lambda n, e: (0, n)),
        ],
        out_specs=[
            pl.BlockSpec((T, TO), lambda n, e: (0, n)),
            pl.BlockSpec((1, 1), lambda n, e: (0, 0)),
        ],
        out_shape=[
            jax.ShapeDtypeStruct((T, D_OUT), jnp.float32),
            jax.ShapeDtypeStruct((1, 1), jnp.float32),
        ],
        scratch_shapes=[
            pltpu.VMEM((SORT, D_IN), jnp.float32),
            pltpu.VMEM((SORT, TO), jnp.float32),
            pltpu.VMEM((T, 1), jnp.int32),
            pltpu.VMEM((1, T), jnp.int32),
            pltpu.SMEM((2 * E,), jnp.int32),
        ],
    )(x, gate_W, gate_b.reshape(1, E), W, b)
    return out, aux[0, 0]


# speedup vs baseline: 1.1731x; 1.1731x over previous
import jax
import jax.numpy as jnp
from jax import lax
from jax.experimental import pallas as pl
from jax.experimental.pallas import tpu as pltpu

T = 256
D_IN = 2048
D_OUT = 2048
E = 8
TM = 64
SORT = 768
TO = 2048
NT = D_OUT // TO


def _moe_body(x_ref, gw_ref, gb_ref, w_ref, b_ref, out_ref, aux_ref,
              xs_scr, os_scr, pc_scr, pr_scr, sm_scr):
    n = pl.program_id(0)
    e = pl.program_id(1)

    @pl.when((n == 0) & (e == 0))
    def _route():
        logits = lax.dot_general(
            x_ref[...], gw_ref[...], (((1,), (1,)), ((), ())),
            preferred_element_type=jnp.float32,
        ) + gb_ref[...]
        m = jnp.max(logits, axis=1, keepdims=True)
        unn = jnp.exp(logits - m)
        p = unn / jnp.sum(unn, axis=1, keepdims=True)
        mean_gate = jnp.mean(p, axis=0, keepdims=True)
        aux_ref[...] = jnp.mean((mean_gate * E) ** 2, axis=1, keepdims=True)

        lane = lax.broadcasted_iota(jnp.int32, (T, E), 1)
        is_max = p == jnp.max(p, axis=1, keepdims=True)
        top1 = jnp.min(jnp.where(is_max, lane, E), axis=1, keepdims=True)
        oh = (lane == top1).astype(jnp.float32)

        counts = jnp.sum(oh, axis=0, keepdims=True)
        pc = jnp.ceil(counts / TM) * TM
        re8 = lax.broadcasted_iota(jnp.int32, (E, E), 0)
        ce8 = lax.broadcasted_iota(jnp.int32, (E, E), 1)
        pc_col = jnp.sum(jnp.where(re8 == ce8, pc, 0.0), axis=1,
                         keepdims=True)
        offs = jnp.sum(jnp.where(re8 < ce8, pc_col, 0.0), axis=0,
                       keepdims=True)
        lane1 = lax.broadcasted_iota(jnp.int32, (1, E), 1)
        for ei in range(E):
            ksc = jnp.sum(jnp.where(lane1 == ei, pc, 0.0)) / TM
            osc = jnp.sum(jnp.where(lane1 == ei, offs, 0.0))
            sm_scr[ei] = ksc.astype(jnp.int32)
            sm_scr[E + ei] = osc.astype(jnp.int32)

        rt = lax.broadcasted_iota(jnp.int32, (T, T), 0)
        ct = lax.broadcasted_iota(jnp.int32, (T, T), 1)
        tril = (rt >= ct).astype(jnp.float32)
        pref = lax.dot_general(tril, oh, (((1,), (0,)), ((), ())),
                               preferred_element_type=jnp.float32)
        rank = jnp.sum(pref * oh, axis=1, keepdims=True) - 1.0
        pos = jnp.sum(oh * offs, axis=1, keepdims=True) + rank
        pc_scr[...] = pos.astype(jnp.int32)
        pos_row = jnp.sum(jnp.where(rt == ct, pos, 0.0), axis=0,
                          keepdims=True).astype(jnp.int32)
        pr_scr[...] = pos_row

        r_iota = lax.broadcasted_iota(jnp.int32, (SORT, T), 0)
        p1 = (r_iota == pos_row).astype(jnp.float32)
        xs_scr[...] = lax.dot_general(p1, x_ref[...],
                                      (((1,), (0,)), ((), ())),
                                      preferred_element_type=jnp.float32)
        os_scr[T:, :] = jnp.zeros((SORT - T, TO), jnp.float32)

    k = sm_scr[e]
    off = sm_scr[E + e]
    r8 = lax.broadcasted_iota(jnp.int32, (E, TO), 0)
    brow = jnp.sum(jnp.where(r8 == e, b_ref[...], 0.0), axis=0, keepdims=True)

    def _subtile(i):
        r0 = pl.multiple_of(off + i * TM, TM)
        xs = xs_scr[pl.ds(r0, TM), :]
        acc = lax.dot_general(xs, w_ref[0], (((1,), (1,)), ((), ())),
                              preferred_element_type=jnp.float32)
        os_scr[pl.ds(r0, TM), :] = acc + brow

    for si in range(2):
        @pl.when(si < k)
        def _static(si=si):
            _subtile(si)

    @pl.when(k > 2)
    def _overflow():
        def mloop(i, _):
            _subtile(i)
            return 0
        lax.fori_loop(2, k, mloop, 0)

    @pl.when(e == E - 1)
    def _combine():
        c_iota = lax.broadcasted_iota(jnp.int32, (T, SORT), 1)
        p2 = (c_iota == pc_scr[...]).astype(jnp.float32)
        out_ref[...] = lax.dot_general(p2, os_scr[...],
                                       (((1,), (0,)), ((), ())),
                                       preferred_element_type=jnp.float32)


def kernel(x, gate_W, gate_b, W, b):
    out, aux = pl.pallas_call(
        _moe_body,
        grid=(NT, E),
        in_specs=[
            pl.BlockSpec((T, D_IN), lambda n, e: (0, 0)),
            pl.BlockSpec((E, D_IN), lambda n, e: (0, 0)),
            pl.BlockSpec((1, E), lambda n, e: (0, 0)),
            pl.BlockSpec((1, TO, D_IN), lambda n, e: (e, n, 0)),
            pl.BlockSpec((E, TO), lambda n, e: (0, n)),
        ],
        out_specs=[
            pl.BlockSpec((T, TO), lambda n, e: (0, n)),
            pl.BlockSpec((1, 1), lambda n, e: (0, 0)),
        ],
        out_shape=[
            jax.ShapeDtypeStruct((T, D_OUT), jnp.float32),
            jax.ShapeDtypeStruct((1, 1), jnp.float32),
        ],
        scratch_shapes=[
            pltpu.VMEM((SORT, D_IN), jnp.float32),
            pltpu.VMEM((SORT, TO), jnp.float32),
            pltpu.VMEM((T, 1), jnp.int32),
            pltpu.VMEM((1, T), jnp.int32),
            pltpu.SMEM((2 * E,), jnp.int32),
        ],
    )(x, gate_W, gate_b.reshape(1, E), W, b)
    return out, aux[0, 0]
